# baseline (speedup 1.0000x reference)
import functools
import math

import jax
import jax.numpy as jnp
from jax import lax
from jax.experimental import pallas as pl
from jax.experimental.pallas import tpu as pltpu
from jax.experimental.pallas import tpu_sc as plsc

D_MODEL = 64
SCALE = math.sqrt(D_MODEL)

NC = 2
NS = 16
NW = NC * NS
C = 128
LANES = 16


def _emb_body(x_hbm, table_hbm, out_hbm, idx_v, rows_v, gsem):
    wid = lax.axis_index("s") * NC + lax.axis_index("c")
    g_per_w = x_hbm.shape[1]
    pltpu.sync_copy(x_hbm.at[wid], idx_v)

    def per_group(g, carry):
        pltpu.async_copy(table_hbm.at[idx_v.at[g]], rows_v, gsem).wait()

        def scale_row(i, c):
            for j in range(D_MODEL // LANES):
                sl = pl.ds(j * LANES, LANES)
                rows_v[i, sl] = rows_v[i, sl] * SCALE
            return c

        lax.fori_loop(0, C, scale_row, 0, unroll=2)
        pltpu.sync_copy(rows_v, out_hbm.at[wid, g])
        return carry

    lax.fori_loop(0, g_per_w, per_group, 0)


def kernel(x, embedding):
    b, h = x.shape
    n = b * h
    assert n % (NW * C) == 0
    g_per_w = n // (NW * C)
    xr = x.reshape(NW, g_per_w, C).astype(jnp.int32)

    mesh = plsc.VectorSubcoreMesh(core_axis_name="c", subcore_axis_name="s")
    out = pl.kernel(
        _emb_body,
        out_type=jax.ShapeDtypeStruct((NW, g_per_w, C, D_MODEL), jnp.float32),
        mesh=mesh,
        scratch_types=[
            pltpu.VMEM((g_per_w, C), jnp.int32),
            pltpu.VMEM((C, D_MODEL), jnp.float32),
            pltpu.SemaphoreType.DMA,
        ],
        compiler_params=pltpu.CompilerParams(use_tc_tiling_on_sc=False),
    )(xr, embedding)
    return out.reshape(b, h, D_MODEL)


# speedup vs baseline: 1.0298x; 1.0298x over previous
import functools
import math

import jax
import jax.numpy as jnp
from jax import lax
from jax.experimental import pallas as pl
from jax.experimental.pallas import tpu as pltpu
from jax.experimental.pallas import tpu_sc as plsc

D_MODEL = 64
SCALE = math.sqrt(D_MODEL)

NC = 2
NS = 16
NW = NC * NS
C = 128
LANES = 16
NBUF = 4


def _emb_body(x_hbm, table_hbm, out_hbm, idx_v, rin, rout, gsems, ssems):
    wid = lax.axis_index("s") * NC + lax.axis_index("c")
    g_tot = x_hbm.shape[1]
    n_steps = g_tot // NBUF
    pltpu.sync_copy(x_hbm.at[wid], idx_v)

    def start_gather(b, g):
        pltpu.async_copy(table_hbm.at[idx_v.at[g]], rin.at[b], gsems.at[b])

    def wait_gather(b, g):
        pltpu.make_async_copy(table_hbm.at[idx_v.at[g]], rin.at[b],
                              gsems.at[b]).wait()

    def start_scatter(b, g):
        pltpu.async_copy(rout.at[b], out_hbm.at[wid, g], ssems.at[b])

    def wait_scatter(b, g):
        pltpu.make_async_copy(rout.at[b], out_hbm.at[wid, g],
                              ssems.at[b]).wait()

    def scale(b):
        def row(i, c):
            for j in range(D_MODEL // LANES):
                sl = pl.ds(j * LANES, LANES)
                rout[b, i, sl] = rin[b, i, sl] * SCALE
            return c

        lax.fori_loop(0, C, row, 0, unroll=4)

    for b in range(NBUF):
        start_gather(b, b)

    for b in range(NBUF):
        wait_gather(b, b)
        scale(b)
        start_gather(b, b + NBUF)
        start_scatter(b, b)

    def step(s, carry):
        g0 = s * NBUF
        for b in range(NBUF):
            g = g0 + b
            wait_gather(b, g)
            wait_scatter(b, g - NBUF)
            scale(b)
            start_gather(b, g + NBUF)
            start_scatter(b, g)
        return carry

    lax.fori_loop(1, n_steps - 1, step, 0)

    g0 = (n_steps - 1) * NBUF
    for b in range(NBUF):
        g = g0 + b
        wait_gather(b, g)
        wait_scatter(b, g - NBUF)
        scale(b)
        start_scatter(b, g)
    for b in range(NBUF):
        wait_scatter(b, g0 + b)


def kernel(x, embedding):
    b, h = x.shape
    n = b * h
    assert n % (NW * C * NBUF) == 0
    g_per_w = n // (NW * C)
    xr = x.reshape(NW, g_per_w, C).astype(jnp.int32)

    mesh = plsc.VectorSubcoreMesh(core_axis_name="c", subcore_axis_name="s")
    out = pl.kernel(
        _emb_body,
        out_type=jax.ShapeDtypeStruct((NW, g_per_w, C, D_MODEL), jnp.float32),
        mesh=mesh,
        scratch_types=[
            pltpu.VMEM((g_per_w, C), jnp.int32),
            pltpu.VMEM((NBUF, C, D_MODEL), jnp.float32),
            pltpu.VMEM((NBUF, C, D_MODEL), jnp.float32),
            pltpu.SemaphoreType.DMA((NBUF,)),
            pltpu.SemaphoreType.DMA((NBUF,)),
        ],
        compiler_params=pltpu.CompilerParams(use_tc_tiling_on_sc=False),
    )(xr, embedding)
    return out.reshape(b, h, D_MODEL)
